# baseline (speedup 1.0000x reference)
import functools

import jax
import jax.numpy as jnp
from jax import lax
from jax.experimental import pallas as pl
from jax.experimental.pallas import tpu as pltpu
from jax.experimental.pallas import tpu_sc as plsc

NC = 2
NS = 16
LANES = 16
CH = 128
RB = 1024
NEG = -1e30


def _cdiv(a, b):
    return (a + b - 1) // b


def _sc_mesh():
    return plsc.VectorSubcoreMesh(core_axis_name="c", subcore_axis_name="s",
                                  num_cores=NC, num_subcores=NS)


def _zero_rows(zbuf, accs, sid, rows_t):
    nfull, rem = divmod(rows_t, CH)
    for k in range(nfull):
        for a in accs:
            pltpu.sync_copy(zbuf, a.at[pl.ds(sid * rows_t + k * CH, CH)])
    if rem:
        for a in accs:
            pltpu.sync_copy(zbuf.at[pl.ds(0, rem)],
                            a.at[pl.ds(sid * rows_t + nfull * CH, rem)])


def _make_sc_agg(n_sn_p, d, chunks_t):
    half = n_sn_p // NC
    rows_t = half // NS
    acc_rows = half + 8
    mesh = _sc_mesh()

    @functools.partial(
        pl.kernel,
        out_type=[jax.ShapeDtypeStruct((NC, NS, rows_t, d), jnp.float32)],
        mesh=mesh,
        scratch_types=[
            pltpu.VMEM_SHARED((acc_rows, d), jnp.float32),
            pltpu.VMEM((chunks_t, CH), jnp.int32),
            pltpu.VMEM((chunks_t, CH), jnp.int32),
            pltpu.VMEM((CH, d), jnp.float32),
            pltpu.VMEM((CH, d), jnp.float32),
            pltpu.VMEM((CH, d), jnp.float32),
            pltpu.SemaphoreType.DMA,
            pltpu.SemaphoreType.DMA,
        ])
    def sc_agg(table, src3d_hbm, dst3d_hbm, agg_out,
               acc, srcb2, dstb2, rows0, rows1, zbuf, sem_g, sem_s):
        cid = lax.axis_index("c")
        sid = lax.axis_index("s")

        def fill_wide(i, _):
            for j in range(d // LANES):
                zbuf[i, pl.ds(j * LANES, LANES)] = jnp.zeros((LANES,), jnp.float32)
            return 0
        lax.fori_loop(0, CH, fill_wide, 0)

        pltpu.sync_copy(src3d_hbm.at[sid], srcb2)
        pltpu.sync_copy(dst3d_hbm.at[sid], dstb2)
        base = cid * half

        def localize(r, _):
            for j in range(CH // LANES):
                dj = dstb2[r, pl.ds(j * LANES, LANES)] - base
                ok = (dj >= 0) & (dj < half)
                dstb2[r, pl.ds(j * LANES, LANES)] = jnp.where(ok, dj, half)
            return 0
        lax.fori_loop(0, chunks_t, localize, 0)

        _zero_rows(zbuf, [acc], sid, rows_t)
        plsc.subcore_barrier()

        rowsb = [rows0, rows1]
        pltpu.async_copy(table.at[srcb2.at[0]], rows0, sem_g)
        pltpu.async_copy(table.at[srcb2.at[1]], rows1, sem_g)

        def body(k2, _):
            for b in range(2):
                k = k2 * 2 + b
                rb = rowsb[b]
                pltpu.make_async_copy(table.at[srcb2.at[0]], rb, sem_g).wait()
                pltpu.async_copy(rb, acc.at[dstb2.at[k]], sem_s, add=True)
                pltpu.make_async_copy(rb, acc.at[dstb2.at[k]], sem_s).wait()

                @pl.when(k + 2 < chunks_t)
                def _():
                    pltpu.async_copy(table.at[srcb2.at[k + 2]], rb, sem_g)
            return 0
        lax.fori_loop(0, chunks_t // 2, body, 0)

        plsc.subcore_barrier()
        pltpu.sync_copy(acc.at[pl.ds(sid * rows_t, rows_t)], agg_out.at[cid, sid])

    return sc_agg


def _make_sc_deg(n_sn_p, d, chunks_t):
    half = n_sn_p // NC
    rows_t = half // NS
    acc_rows = half + 8
    mesh = _sc_mesh()
    fire = 8

    @functools.partial(
        pl.kernel,
        out_type=[jax.ShapeDtypeStruct((NC, NS, rows_t, d), jnp.float32)],
        mesh=mesh,
        scratch_types=[
            pltpu.VMEM_SHARED((acc_rows, d), jnp.float32),
            pltpu.VMEM((chunks_t, CH), jnp.int32),
            pltpu.VMEM((CH, d), jnp.float32),
            pltpu.VMEM((CH, d), jnp.float32),
            pltpu.SemaphoreType.DMA,
        ])
    def sc_deg(dst3d_hbm, deg_out, dacc, dstb2, onesb, zbuf, sem_s):
        cid = lax.axis_index("c")
        sid = lax.axis_index("s")

        def fill_wide(i, _):
            for j in range(d // LANES):
                zbuf[i, pl.ds(j * LANES, LANES)] = jnp.zeros((LANES,), jnp.float32)
                onesb[i, pl.ds(j * LANES, LANES)] = jnp.ones((LANES,), jnp.float32)
            return 0
        lax.fori_loop(0, CH, fill_wide, 0)

        pltpu.sync_copy(dst3d_hbm.at[sid], dstb2)
        base = cid * half

        def localize(r, _):
            for j in range(CH // LANES):
                dj = dstb2[r, pl.ds(j * LANES, LANES)] - base
                ok = (dj >= 0) & (dj < half)
                dstb2[r, pl.ds(j * LANES, LANES)] = jnp.where(ok, dj, half)
            return 0
        lax.fori_loop(0, chunks_t, localize, 0)

        _zero_rows(zbuf, [dacc], sid, rows_t)
        plsc.subcore_barrier()

        def body(kg, _):
            for b in range(fire):
                k = kg * fire + b
                pltpu.async_copy(onesb, dacc.at[dstb2.at[k]], sem_s, add=True)
            for b in range(fire):
                k = kg * fire + b
                pltpu.make_async_copy(onesb, dacc.at[dstb2.at[k]], sem_s).wait()
            return 0
        lax.fori_loop(0, chunks_t // fire, body, 0)

        plsc.subcore_barrier()
        pltpu.sync_copy(dacc.at[pl.ds(sid * rows_t, rows_t)], deg_out.at[cid, sid])

    return sc_deg


def _build_av(srcv2d_ref, dstv2d_ref, n_v_p):
    nodes = lax.broadcasted_iota(jnp.int32, (n_v_p, 1), 0)
    av = jnp.zeros((n_v_p, n_v_p), jnp.float32)
    for r in range(srcv2d_ref.shape[0]):
        srow = srcv2d_ref[r, :].reshape(1, -1)
        drow = dstv2d_ref[r, :].reshape(1, -1)
        ohs = (nodes == srow).astype(jnp.float32)
        ohd = (nodes == drow).astype(jnp.float32)
        av = av + lax.dot_general(ohd, ohs, (((1,), (1,)), ((), ())),
                                  preferred_element_type=jnp.float32)
    return av


def _tc1_body(n_v_p, agg_ref, deg_ref, x_ref, w1_ref, b1_ref, w2_ref,
              srcv2d_ref, dstv2d_ref, xv_ref, w1v_ref, b1v_ref, w2v_ref,
              p_ref, pv_ref, av_ref):
    deg = jnp.maximum(deg_ref[:, 0:1], 1.0)
    pre = agg_ref[...] / deg + x_ref[...]
    h = jnp.maximum(
        jnp.dot(pre, w1_ref[...], preferred_element_type=jnp.float32) + b1_ref[...],
        0.0)
    p_ref[...] = jnp.dot(h, w2_ref[...], preferred_element_type=jnp.float32)

    @pl.when(pl.program_id(0) == 0)
    def _():
        av = _build_av(srcv2d_ref, dstv2d_ref, n_v_p)
        av_ref[...] = av
        degv = jnp.maximum(
            jnp.dot(av, jnp.ones((n_v_p, 1), jnp.float32),
                    preferred_element_type=jnp.float32), 1.0)
        aggv = jnp.dot(av, xv_ref[...], preferred_element_type=jnp.float32)
        prev = aggv / degv + xv_ref[...]
        hv = jnp.maximum(
            jnp.dot(prev, w1v_ref[...], preferred_element_type=jnp.float32) + b1v_ref[...],
            0.0)
        pv_ref[...] = jnp.dot(hv, w2v_ref[...], preferred_element_type=jnp.float32)


def _tc2_body(n_sn, n_v, nsb,
              idx_ref, agg2_ref, deg_ref, p_ref, b2_ref, watt_ref,
              av_ref, pv_ref, b2v_ref, wattv_ref,
              l1_ref, bl1_ref, l2_ref, bl2_ref,
              out_ref, vec_ref, ms_ref):
    i = pl.program_id(0)

    @pl.when(i == 0)
    def _():
        ms_ref[0] = NEG
        ms_ref[1] = 0.0
        vec_ref[...] = jnp.zeros_like(vec_ref)

    @pl.when(i < nsb)
    def _():
        deg = jnp.maximum(deg_ref[:, 0:1], 1.0)
        h = agg2_ref[...] / deg + p_ref[...] + b2_ref[...]
        rows = i * RB + lax.broadcasted_iota(jnp.int32, (RB, 1), 0)
        valid = rows < n_sn
        h = jnp.where(valid, h, 0.0)
        e = jnp.tanh(jnp.dot(h, watt_ref[...], preferred_element_type=jnp.float32))
        e = jnp.where(valid, e, NEG)
        m_old = ms_ref[0]
        s_old = ms_ref[1]
        m_new = jnp.maximum(m_old, jnp.max(e))
        corr = jnp.exp(m_old - m_new)
        w = jnp.exp(e - m_new)
        ms_ref[0] = m_new
        ms_ref[1] = s_old * corr + jnp.sum(w)
        vec_ref[...] = vec_ref[...] * corr + lax.dot_general(
            w, h, (((0,), (0,)), ((), ())), preferred_element_type=jnp.float32)

    @pl.when(i == nsb)
    def _():
        sn_pool = vec_ref[...] / ms_ref[1]
        av = av_ref[...]
        n_v_p = av.shape[0]
        degv = jnp.maximum(
            jnp.dot(av, jnp.ones((n_v_p, 1), jnp.float32),
                    preferred_element_type=jnp.float32), 1.0)
        agg2v = jnp.dot(av, pv_ref[...], preferred_element_type=jnp.float32)
        hv = agg2v / degv + pv_ref[...] + b2v_ref[...]
        rowsv = lax.broadcasted_iota(jnp.int32, (n_v_p, 1), 0)
        validv = rowsv < n_v
        hv = jnp.where(validv, hv, 0.0)
        iv = idx_ref[0, 0]
        q = jnp.sum(jnp.where(rowsv == iv, hv, 0.0), axis=0, keepdims=True)
        t = jnp.tanh(jnp.dot(hv, wattv_ref[...], preferred_element_type=jnp.float32))
        ev = lax.dot_general(t, q, (((1,), (1,)), ((), ())),
                             preferred_element_type=jnp.float32)
        ev = jnp.where(validv, ev, NEG)
        mv = jnp.max(ev)
        wv = jnp.exp(ev - mv)
        v_pool = lax.dot_general(wv, hv, (((0,), (0,)), ((), ())),
                                 preferred_element_type=jnp.float32) / jnp.sum(wv)
        state = jnp.concatenate([sn_pool, v_pool], axis=1)
        s1 = jnp.maximum(
            jnp.dot(state, l1_ref[...], preferred_element_type=jnp.float32) + bl1_ref[...],
            0.0)
        out_ref[...] = jnp.maximum(
            jnp.dot(s1, l2_ref[...], preferred_element_type=jnp.float32) + bl2_ref[...],
            0.0)


def _tc3_body(s2_ref, l3_ref, bl3_ref, out_ref):
    out_ref[...] = jnp.dot(s2_ref[...], l3_ref[...],
                           preferred_element_type=jnp.float32) + bl3_ref[...]


def kernel(x_sn, edge_index_sn, x_vnr, edge_index_vnr, idx,
           W1_sn, b1_sn, W2_sn, b2_sn,
           W1_v, b1_v, W2_v, b2_v,
           w_att_sn, W_att_v,
           L1, bL1, L2, bL2, L3, bL3):
    f32 = jnp.float32
    i32 = jnp.int32
    n_sn, d_sn = x_sn.shape
    n_v, d_v = x_vnr.shape
    e_sn = edge_index_sn.shape[1]
    e_v = edge_index_vnr.shape[1]
    h_dim = W1_sn.shape[1]
    g_dim = W2_sn.shape[1]
    a_dim = L3.shape[1]

    n_sn_p = _cdiv(n_sn, NC * NS) * (NC * NS)
    n_v_p = _cdiv(n_v + 1, 8) * 8
    nsb = _cdiv(n_sn_p, RB)

    chunks_t = _cdiv(_cdiv(_cdiv(e_sn, CH), NS), 8) * 8
    e_sn_p = NS * chunks_t * CH
    e_v_p = _cdiv(e_v, CH) * CH

    src_s3d = jnp.concatenate([edge_index_sn[0].astype(i32),
                               jnp.zeros((e_sn_p - e_sn,), i32)]
                              ).reshape(NS, chunks_t, CH)
    dst_s3d = jnp.concatenate([edge_index_sn[1].astype(i32),
                               jnp.full((e_sn_p - e_sn,), n_sn, i32)]
                              ).reshape(NS, chunks_t, CH)
    src_v2d = jnp.concatenate([edge_index_vnr[0].astype(i32),
                               jnp.zeros((e_v_p - e_v,), i32)]).reshape(-1, CH)
    dst_v2d = jnp.concatenate([edge_index_vnr[1].astype(i32),
                               jnp.full((e_v_p - e_v,), n_v, i32)]).reshape(-1, CH)
    nvr = e_v_p // CH

    xv_p = jnp.pad(x_vnr, ((0, n_v_p - n_v), (0, d_sn - d_v)))
    w1v_p = jnp.pad(W1_v, ((0, d_sn - d_v), (0, 0)))

    idx_arr = jnp.asarray(idx, i32).reshape(1, 1)
    b1 = b1_sn.reshape(1, h_dim)
    b2 = b2_sn.reshape(1, g_dim)
    b1v = b1_v.reshape(1, h_dim)
    b2v = b2_v.reshape(1, g_dim)
    watt = w_att_sn.reshape(g_dim, 1)
    bl1 = bL1.reshape(1, -1)
    bl2 = bL2.reshape(1, -1)
    bl3 = bL3.reshape(1, -1)

    sc1 = _make_sc_agg(n_sn_p, d_sn, chunks_t)
    agg1 = sc1(x_sn, src_s3d, dst_s3d)[0].reshape(n_sn_p, d_sn)
    scdeg = _make_sc_deg(n_sn_p, d_sn, chunks_t)
    deg = scdeg(dst_s3d)[0].reshape(n_sn_p, d_sn)

    p, pv, av = pl.pallas_call(
        functools.partial(_tc1_body, n_v_p),
        grid=(nsb,),
        in_specs=[
            pl.BlockSpec((RB, d_sn), lambda i: (i, 0)),
            pl.BlockSpec((RB, d_sn), lambda i: (i, 0)),
            pl.BlockSpec((RB, d_sn), lambda i: (i, 0)),
            pl.BlockSpec((d_sn, h_dim), lambda i: (0, 0)),
            pl.BlockSpec((1, h_dim), lambda i: (0, 0)),
            pl.BlockSpec((h_dim, g_dim), lambda i: (0, 0)),
            pl.BlockSpec((nvr, CH), lambda i: (0, 0)),
            pl.BlockSpec((nvr, CH), lambda i: (0, 0)),
            pl.BlockSpec((n_v_p, d_sn), lambda i: (0, 0)),
            pl.BlockSpec((d_sn, h_dim), lambda i: (0, 0)),
            pl.BlockSpec((1, h_dim), lambda i: (0, 0)),
            pl.BlockSpec((h_dim, g_dim), lambda i: (0, 0)),
        ],
        out_specs=[
            pl.BlockSpec((RB, g_dim), lambda i: (i, 0)),
            pl.BlockSpec((n_v_p, g_dim), lambda i: (0, 0)),
            pl.BlockSpec((n_v_p, n_v_p), lambda i: (0, 0)),
        ],
        out_shape=[
            jax.ShapeDtypeStruct((n_sn, g_dim), f32),
            jax.ShapeDtypeStruct((n_v_p, g_dim), f32),
            jax.ShapeDtypeStruct((n_v_p, n_v_p), f32),
        ],
    )(agg1, deg, x_sn, W1_sn, b1, W2_sn, src_v2d, dst_v2d, xv_p, w1v_p, b1v, W2_v)

    sc2 = _make_sc_agg(n_sn_p, g_dim, chunks_t)
    agg2 = sc2(p, src_s3d, dst_s3d)[0].reshape(n_sn_p, g_dim)

    def capped(i):
        return jnp.minimum(i, nsb - 1)

    s2 = pl.pallas_call(
        functools.partial(_tc2_body, n_sn, n_v, nsb),
        grid=(nsb + 1,),
        in_specs=[
            pl.BlockSpec(memory_space=pltpu.SMEM),
            pl.BlockSpec((RB, g_dim), lambda i: (capped(i), 0)),
            pl.BlockSpec((RB, d_sn), lambda i: (capped(i), 0)),
            pl.BlockSpec((RB, g_dim), lambda i: (capped(i), 0)),
            pl.BlockSpec((1, g_dim), lambda i: (0, 0)),
            pl.BlockSpec((g_dim, 1), lambda i: (0, 0)),
            pl.BlockSpec((n_v_p, n_v_p), lambda i: (0, 0)),
            pl.BlockSpec((n_v_p, g_dim), lambda i: (0, 0)),
            pl.BlockSpec((1, g_dim), lambda i: (0, 0)),
            pl.BlockSpec((g_dim, g_dim), lambda i: (0, 0)),
            pl.BlockSpec((2 * g_dim, h_dim), lambda i: (0, 0)),
            pl.BlockSpec((1, h_dim), lambda i: (0, 0)),
            pl.BlockSpec((h_dim, h_dim), lambda i: (0, 0)),
            pl.BlockSpec((1, h_dim), lambda i: (0, 0)),
        ],
        out_specs=pl.BlockSpec((1, h_dim), lambda i: (0, 0)),
        out_shape=jax.ShapeDtypeStruct((1, h_dim), f32),
        scratch_shapes=[
            pltpu.VMEM((1, g_dim), f32),
            pltpu.SMEM((2,), f32),
        ],
    )(idx_arr, agg2, deg, p, b2, watt, av, pv, b2v, W_att_v,
      L1, bl1, L2, bl2)

    cb = 2048
    values = pl.pallas_call(
        _tc3_body,
        grid=(_cdiv(a_dim, cb),),
        in_specs=[
            pl.BlockSpec((1, h_dim), lambda i: (0, 0)),
            pl.BlockSpec((h_dim, cb), lambda i: (0, i)),
            pl.BlockSpec((1, cb), lambda i: (0, i)),
        ],
        out_specs=pl.BlockSpec((1, cb), lambda i: (0, i)),
        out_shape=jax.ShapeDtypeStruct((1, a_dim), f32),
    )(s2, L3, bl3)

    return values


# speedup vs baseline: 1.4560x; 1.4560x over previous
import functools

import jax
import jax.numpy as jnp
from jax import lax
from jax.experimental import pallas as pl
from jax.experimental.pallas import tpu as pltpu
from jax.experimental.pallas import tpu_sc as plsc

NC = 2
NS = 16
LANES = 16
CH = 128
RB = 1024
NEG = -1e30


def _cdiv(a, b):
    return (a + b - 1) // b


def _make_sc_agg(n_sn_p, d, sn_chunks_t, with_deg):
    half = n_sn_p // NC
    rows_t = half // NS
    acc_rows = half + 8
    mesh = plsc.VectorSubcoreMesh(core_axis_name="c", subcore_axis_name="s",
                                  num_cores=NC, num_subcores=NS)

    out_type = [jax.ShapeDtypeStruct((NC, NS, rows_t, d), jnp.float32)]
    scratch = [
        pltpu.VMEM_SHARED((acc_rows, d), jnp.float32),
        pltpu.VMEM((CH,), jnp.int32),
        pltpu.VMEM((CH,), jnp.int32),
        pltpu.VMEM((CH, d), jnp.float32),
        pltpu.VMEM((CH, d), jnp.float32),
    ]
    if with_deg:
        out_type += [jax.ShapeDtypeStruct((NC, NS, rows_t, d), jnp.float32)]
        scratch += [
            pltpu.VMEM_SHARED((acc_rows, d), jnp.float32),
            pltpu.VMEM((CH, d), jnp.float32),
        ]
    scratch += [pltpu.SemaphoreType.DMA]

    @functools.partial(pl.kernel, out_type=out_type, mesh=mesh,
                       scratch_types=scratch)
    def sc_agg(table, src_hbm, dst_hbm, *rest):
        if with_deg:
            (agg_out, deg_out,
             acc, srcb, dstb, rowsb, zbuf, dacc, onesb, sem) = rest
        else:
            (agg_out, acc, srcb, dstb, rowsb, zbuf, sem) = rest

        cid = lax.axis_index("c")
        sid = lax.axis_index("s")

        def fill_wide(i, _):
            for j in range(d // LANES):
                zbuf[i, pl.ds(j * LANES, LANES)] = jnp.zeros((LANES,), jnp.float32)
                if with_deg:
                    onesb[i, pl.ds(j * LANES, LANES)] = jnp.ones((LANES,), jnp.float32)
            return 0
        lax.fori_loop(0, CH, fill_wide, 0)

        nfull, rem = divmod(rows_t, CH)
        for k in range(nfull):
            pltpu.sync_copy(zbuf, acc.at[pl.ds(sid * rows_t + k * CH, CH)])
            if with_deg:
                pltpu.sync_copy(zbuf, dacc.at[pl.ds(sid * rows_t + k * CH, CH)])
        if rem:
            pltpu.sync_copy(zbuf.at[pl.ds(0, rem)],
                            acc.at[pl.ds(sid * rows_t + nfull * CH, rem)])
            if with_deg:
                pltpu.sync_copy(zbuf.at[pl.ds(0, rem)],
                                dacc.at[pl.ds(sid * rows_t + nfull * CH, rem)])
        plsc.subcore_barrier()

        base = cid * half

        def body(k, _):
            e0 = (sid * sn_chunks_t + k) * CH
            pltpu.sync_copy(src_hbm.at[pl.ds(e0, CH)], srcb)
            pltpu.sync_copy(dst_hbm.at[pl.ds(e0, CH)], dstb)
            gat = pltpu.async_copy(table.at[srcb], rowsb, sem)
            for j in range(CH // LANES):
                dj = dstb[pl.ds(j * LANES, LANES)] - base
                ok = (dj >= 0) & (dj < half)
                dstb[pl.ds(j * LANES, LANES)] = jnp.where(ok, dj, half)
            gat.wait()
            if with_deg:
                s1 = pltpu.async_copy(rowsb, acc.at[dstb], sem, add=True)
                s2 = pltpu.async_copy(onesb, dacc.at[dstb], sem, add=True)
                s1.wait()
                s2.wait()
            else:
                pltpu.sync_copy(rowsb, acc.at[dstb], add=True)
            return 0
        lax.fori_loop(0, sn_chunks_t, body, 0)

        plsc.subcore_barrier()

        pltpu.sync_copy(acc.at[pl.ds(sid * rows_t, rows_t)], agg_out.at[cid, sid])
        if with_deg:
            pltpu.sync_copy(dacc.at[pl.ds(sid * rows_t, rows_t)], deg_out.at[cid, sid])

    return sc_agg


def _build_av(srcv2d_ref, dstv2d_ref, n_v_p):
    nodes = lax.broadcasted_iota(jnp.int32, (n_v_p, 1), 0)
    av = jnp.zeros((n_v_p, n_v_p), jnp.float32)
    for r in range(srcv2d_ref.shape[0]):
        srow = srcv2d_ref[r, :].reshape(1, -1)
        drow = dstv2d_ref[r, :].reshape(1, -1)
        ohs = (nodes == srow).astype(jnp.float32)
        ohd = (nodes == drow).astype(jnp.float32)
        av = av + lax.dot_general(ohd, ohs, (((1,), (1,)), ((), ())),
                                  preferred_element_type=jnp.float32)
    return av


def _tc1_body(n_v_p, agg_ref, deg_ref, x_ref, w1_ref, b1_ref, w2_ref,
              srcv2d_ref, dstv2d_ref, xv_ref, w1v_ref, b1v_ref, w2v_ref,
              p_ref, pv_ref, av_ref):
    deg = jnp.maximum(deg_ref[:, 0:1], 1.0)
    pre = agg_ref[...] / deg + x_ref[...]
    h = jnp.maximum(
        jnp.dot(pre, w1_ref[...], preferred_element_type=jnp.float32) + b1_ref[...],
        0.0)
    p_ref[...] = jnp.dot(h, w2_ref[...], preferred_element_type=jnp.float32)

    @pl.when(pl.program_id(0) == 0)
    def _():
        av = _build_av(srcv2d_ref, dstv2d_ref, n_v_p)
        av_ref[...] = av
        degv = jnp.maximum(
            jnp.dot(av, jnp.ones((n_v_p, 1), jnp.float32),
                    preferred_element_type=jnp.float32), 1.0)
        aggv = jnp.dot(av, xv_ref[...], preferred_element_type=jnp.float32)
        prev = aggv / degv + xv_ref[...]
        hv = jnp.maximum(
            jnp.dot(prev, w1v_ref[...], preferred_element_type=jnp.float32) + b1v_ref[...],
            0.0)
        pv_ref[...] = jnp.dot(hv, w2v_ref[...], preferred_element_type=jnp.float32)


def _tc2_body(n_sn, n_v, nsb,
              idx_ref, agg2_ref, deg_ref, p_ref, b2_ref, watt_ref,
              av_ref, pv_ref, b2v_ref, wattv_ref,
              l1_ref, bl1_ref, l2_ref, bl2_ref,
              out_ref, vec_ref, ms_ref):
    i = pl.program_id(0)

    @pl.when(i == 0)
    def _():
        ms_ref[0] = NEG
        ms_ref[1] = 0.0
        vec_ref[...] = jnp.zeros_like(vec_ref)

    @pl.when(i < nsb)
    def _():
        deg = jnp.maximum(deg_ref[:, 0:1], 1.0)
        h = agg2_ref[...] / deg + p_ref[...] + b2_ref[...]
        rows = i * RB + lax.broadcasted_iota(jnp.int32, (RB, 1), 0)
        valid = rows < n_sn
        h = jnp.where(valid, h, 0.0)
        e = jnp.tanh(jnp.dot(h, watt_ref[...], preferred_element_type=jnp.float32))
        e = jnp.where(valid, e, NEG)
        m_old = ms_ref[0]
        s_old = ms_ref[1]
        m_new = jnp.maximum(m_old, jnp.max(e))
        corr = jnp.exp(m_old - m_new)
        w = jnp.exp(e - m_new)
        ms_ref[0] = m_new
        ms_ref[1] = s_old * corr + jnp.sum(w)
        vec_ref[...] = vec_ref[...] * corr + lax.dot_general(
            w, h, (((0,), (0,)), ((), ())), preferred_element_type=jnp.float32)

    @pl.when(i == nsb)
    def _():
        sn_pool = vec_ref[...] / ms_ref[1]
        av = av_ref[...]
        n_v_p = av.shape[0]
        degv = jnp.maximum(
            jnp.dot(av, jnp.ones((n_v_p, 1), jnp.float32),
                    preferred_element_type=jnp.float32), 1.0)
        agg2v = jnp.dot(av, pv_ref[...], preferred_element_type=jnp.float32)
        hv = agg2v / degv + pv_ref[...] + b2v_ref[...]
        rowsv = lax.broadcasted_iota(jnp.int32, (n_v_p, 1), 0)
        validv = rowsv < n_v
        hv = jnp.where(validv, hv, 0.0)
        iv = idx_ref[0, 0]
        q = jnp.sum(jnp.where(rowsv == iv, hv, 0.0), axis=0, keepdims=True)
        t = jnp.tanh(jnp.dot(hv, wattv_ref[...], preferred_element_type=jnp.float32))
        ev = lax.dot_general(t, q, (((1,), (1,)), ((), ())),
                             preferred_element_type=jnp.float32)
        ev = jnp.where(validv, ev, NEG)
        mv = jnp.max(ev)
        wv = jnp.exp(ev - mv)
        v_pool = lax.dot_general(wv, hv, (((0,), (0,)), ((), ())),
                                 preferred_element_type=jnp.float32) / jnp.sum(wv)
        state = jnp.concatenate([sn_pool, v_pool], axis=1)
        s1 = jnp.maximum(
            jnp.dot(state, l1_ref[...], preferred_element_type=jnp.float32) + bl1_ref[...],
            0.0)
        out_ref[...] = jnp.maximum(
            jnp.dot(s1, l2_ref[...], preferred_element_type=jnp.float32) + bl2_ref[...],
            0.0)


def _tc3_body(s2_ref, l3_ref, bl3_ref, out_ref):
    out_ref[...] = jnp.dot(s2_ref[...], l3_ref[...],
                           preferred_element_type=jnp.float32) + bl3_ref[...]


def kernel(x_sn, edge_index_sn, x_vnr, edge_index_vnr, idx,
           W1_sn, b1_sn, W2_sn, b2_sn,
           W1_v, b1_v, W2_v, b2_v,
           w_att_sn, W_att_v,
           L1, bL1, L2, bL2, L3, bL3):
    f32 = jnp.float32
    i32 = jnp.int32
    n_sn, d_sn = x_sn.shape
    n_v, d_v = x_vnr.shape
    e_sn = edge_index_sn.shape[1]
    e_v = edge_index_vnr.shape[1]
    h_dim = W1_sn.shape[1]
    g_dim = W2_sn.shape[1]
    a_dim = L3.shape[1]

    n_sn_p = _cdiv(n_sn, NC * NS) * (NC * NS)
    n_v_p = _cdiv(n_v + 1, 8) * 8
    nsb = _cdiv(n_sn_p, RB)

    sn_chunks_t = _cdiv(_cdiv(e_sn, CH), NS)
    e_sn_p = NS * sn_chunks_t * CH
    e_v_p = _cdiv(e_v, CH) * CH

    src_s = jnp.concatenate([edge_index_sn[0].astype(i32),
                             jnp.zeros((e_sn_p - e_sn,), i32)])
    dst_s = jnp.concatenate([edge_index_sn[1].astype(i32),
                             jnp.full((e_sn_p - e_sn,), n_sn, i32)])
    src_v2d = jnp.concatenate([edge_index_vnr[0].astype(i32),
                               jnp.zeros((e_v_p - e_v,), i32)]).reshape(-1, CH)
    dst_v2d = jnp.concatenate([edge_index_vnr[1].astype(i32),
                               jnp.full((e_v_p - e_v,), n_v, i32)]).reshape(-1, CH)
    nvr = e_v_p // CH

    xv_p = jnp.pad(x_vnr, ((0, n_v_p - n_v), (0, d_sn - d_v)))
    w1v_p = jnp.pad(W1_v, ((0, d_sn - d_v), (0, 0)))

    idx_arr = jnp.asarray(idx, i32).reshape(1, 1)
    b1 = b1_sn.reshape(1, h_dim)
    b2 = b2_sn.reshape(1, g_dim)
    b1v = b1_v.reshape(1, h_dim)
    b2v = b2_v.reshape(1, g_dim)
    watt = w_att_sn.reshape(g_dim, 1)
    bl1 = bL1.reshape(1, -1)
    bl2 = bL2.reshape(1, -1)
    bl3 = bL3.reshape(1, -1)

    sc1 = _make_sc_agg(n_sn_p, d_sn, sn_chunks_t, with_deg=True)
    agg1, deg = sc1(x_sn, src_s, dst_s)
    agg1 = agg1.reshape(n_sn_p, d_sn)
    deg = deg.reshape(n_sn_p, d_sn)

    p, pv, av = pl.pallas_call(
        functools.partial(_tc1_body, n_v_p),
        grid=(nsb,),
        in_specs=[
            pl.BlockSpec((RB, d_sn), lambda i: (i, 0)),
            pl.BlockSpec((RB, d_sn), lambda i: (i, 0)),
            pl.BlockSpec((RB, d_sn), lambda i: (i, 0)),
            pl.BlockSpec((d_sn, h_dim), lambda i: (0, 0)),
            pl.BlockSpec((1, h_dim), lambda i: (0, 0)),
            pl.BlockSpec((h_dim, g_dim), lambda i: (0, 0)),
            pl.BlockSpec((nvr, CH), lambda i: (0, 0)),
            pl.BlockSpec((nvr, CH), lambda i: (0, 0)),
            pl.BlockSpec((n_v_p, d_sn), lambda i: (0, 0)),
            pl.BlockSpec((d_sn, h_dim), lambda i: (0, 0)),
            pl.BlockSpec((1, h_dim), lambda i: (0, 0)),
            pl.BlockSpec((h_dim, g_dim), lambda i: (0, 0)),
        ],
        out_specs=[
            pl.BlockSpec((RB, g_dim), lambda i: (i, 0)),
            pl.BlockSpec((n_v_p, g_dim), lambda i: (0, 0)),
            pl.BlockSpec((n_v_p, n_v_p), lambda i: (0, 0)),
        ],
        out_shape=[
            jax.ShapeDtypeStruct((n_sn, g_dim), f32),
            jax.ShapeDtypeStruct((n_v_p, g_dim), f32),
            jax.ShapeDtypeStruct((n_v_p, n_v_p), f32),
        ],
    )(agg1, deg, x_sn, W1_sn, b1, W2_sn, src_v2d, dst_v2d, xv_p, w1v_p, b1v, W2_v)

    sc2 = _make_sc_agg(n_sn_p, g_dim, sn_chunks_t, with_deg=False)
    agg2 = sc2(p, src_s, dst_s)[0].reshape(n_sn_p, g_dim)

    def capped(i):
        return jnp.minimum(i, nsb - 1)

    s2 = pl.pallas_call(
        functools.partial(_tc2_body, n_sn, n_v, nsb),
        grid=(nsb + 1,),
        in_specs=[
            pl.BlockSpec(memory_space=pltpu.SMEM),
            pl.BlockSpec((RB, g_dim), lambda i: (capped(i), 0)),
            pl.BlockSpec((RB, d_sn), lambda i: (capped(i), 0)),
            pl.BlockSpec((RB, g_dim), lambda i: (capped(i), 0)),
            pl.BlockSpec((1, g_dim), lambda i: (0, 0)),
            pl.BlockSpec((g_dim, 1), lambda i: (0, 0)),
            pl.BlockSpec((n_v_p, n_v_p), lambda i: (0, 0)),
            pl.BlockSpec((n_v_p, g_dim), lambda i: (0, 0)),
            pl.BlockSpec((1, g_dim), lambda i: (0, 0)),
            pl.BlockSpec((g_dim, g_dim), lambda i: (0, 0)),
            pl.BlockSpec((2 * g_dim, h_dim), lambda i: (0, 0)),
            pl.BlockSpec((1, h_dim), lambda i: (0, 0)),
            pl.BlockSpec((h_dim, h_dim), lambda i: (0, 0)),
            pl.BlockSpec((1, h_dim), lambda i: (0, 0)),
        ],
        out_specs=pl.BlockSpec((1, h_dim), lambda i: (0, 0)),
        out_shape=jax.ShapeDtypeStruct((1, h_dim), f32),
        scratch_shapes=[
            pltpu.VMEM((1, g_dim), f32),
            pltpu.SMEM((2,), f32),
        ],
    )(idx_arr, agg2, deg, p, b2, watt, av, pv, b2v, W_att_v,
      L1, bl1, L2, bl2)

    cb = 2048
    values = pl.pallas_call(
        _tc3_body,
        grid=(_cdiv(a_dim, cb),),
        in_specs=[
            pl.BlockSpec((1, h_dim), lambda i: (0, 0)),
            pl.BlockSpec((h_dim, cb), lambda i: (0, i)),
            pl.BlockSpec((1, cb), lambda i: (0, i)),
        ],
        out_specs=pl.BlockSpec((1, cb), lambda i: (0, i)),
        out_shape=jax.ShapeDtypeStruct((1, a_dim), f32),
    )(s2, L3, bl3)

    return values
